# Initial kernel scaffold; baseline (speedup 1.0000x reference)
import jax
import jax.numpy as jnp
from jax.experimental import pallas as pl


def kernel(x, edge_index, batch, W1, b1, W2, b2, Wm1, bm1, Wm2, bm2):
    raise NotImplementedError("write your pallas kernel here")


# speedup vs baseline: 14.8754x; 14.8754x over previous
import functools

import jax
import jax.numpy as jnp
from jax import lax
from jax.experimental import pallas as pl
from jax.experimental.pallas import tpu as pltpu
from jax.experimental.pallas import tpu_sc as plsc

_N = 10000
_E = 320000
_IN = 128
_HID = 64
_OUTD = 128
_G = 256

_NP = 10240
_NC = 2
_NS = 16
_NW = _NC * _NS
_CH = 128
_NCH = 79
_EPT = _CH * _NCH
_EP = _EPT * _NW
_RPT = _NP // _NS
_BLK = 512
_NB = _NP // _BLK
_DISCARD = _N + 128


def _deg_body(dst_hbm, out_hbm, dst_v, ones_v, zero_v, acc):
    cid = lax.axis_index("c")
    sid = lax.axis_index("s")
    wid = sid * _NC + cid

    def _init(i, carry):
        ones_v[i, :] = jnp.ones((16,), jnp.float32)
        zero_v[i, :] = jnp.zeros((16,), jnp.float32)
        return carry

    lax.fori_loop(0, _CH, _init, 0)
    for k in range(_RPT // _CH):
        pltpu.sync_copy(zero_v, acc.at[pl.ds(sid * _RPT + k * _CH, _CH)])
    plsc.subcore_barrier()

    def _body(j, carry):
        off = wid * _EPT + j * _CH
        pltpu.sync_copy(dst_hbm.at[pl.ds(off, _CH)], dst_v)
        pltpu.sync_copy(ones_v, acc.at[dst_v], add=True)
        return carry

    lax.fori_loop(0, _NCH, _body, 0)
    plsc.subcore_barrier()
    pltpu.sync_copy(acc.at[pl.ds(sid * _RPT, _RPT)],
                    out_hbm.at[cid].at[pl.ds(sid * _RPT, _RPT)])


def _scatter_body(src_hbm, dst_hbm, y_hbm, out_hbm,
                  src_v, dst_v, rows_v, zero_v, acc, sem):
    cid = lax.axis_index("c")
    sid = lax.axis_index("s")
    wid = sid * _NC + cid

    def _init(i, carry):
        z = jnp.zeros((16,), jnp.float32)
        for j in range(_HID // 16):
            zero_v[i, pl.ds(j * 16, 16)] = z
        return carry

    lax.fori_loop(0, _CH, _init, 0)
    for k in range(_RPT // _CH):
        pltpu.sync_copy(zero_v, acc.at[pl.ds(sid * _RPT + k * _CH, _CH)])
    plsc.subcore_barrier()

    def _body(j, carry):
        off = wid * _EPT + j * _CH
        pltpu.sync_copy(src_hbm.at[pl.ds(off, _CH)], src_v)
        pltpu.sync_copy(dst_hbm.at[pl.ds(off, _CH)], dst_v)
        pltpu.async_copy(y_hbm.at[src_v], rows_v, sem).wait()
        pltpu.sync_copy(rows_v, acc.at[dst_v], add=True)
        return carry

    lax.fori_loop(0, _NCH, _body, 0)
    plsc.subcore_barrier()
    pltpu.sync_copy(acc.at[pl.ds(sid * _RPT, _RPT)],
                    out_hbm.at[cid].at[pl.ds(sid * _RPT, _RPT)])


@functools.cache
def _sc_kernels():
    mesh = plsc.VectorSubcoreMesh(
        core_axis_name="c", subcore_axis_name="s")
    params = pltpu.CompilerParams(use_tc_tiling_on_sc=False)
    deg = pl.kernel(
        _deg_body,
        mesh=mesh,
        compiler_params=params,
        out_type=jax.ShapeDtypeStruct((_NC, _NP, 16), jnp.float32),
        scratch_types=[
            pltpu.VMEM((_CH,), jnp.int32),
            pltpu.VMEM((_CH, 16), jnp.float32),
            pltpu.VMEM((_CH, 16), jnp.float32),
            pltpu.VMEM_SHARED((_NP, 16), jnp.float32),
        ],
    )
    scatter = pl.kernel(
        _scatter_body,
        mesh=mesh,
        compiler_params=params,
        out_type=jax.ShapeDtypeStruct((_NC, _NP, _HID), jnp.float32),
        scratch_types=[
            pltpu.VMEM((_CH,), jnp.int32),
            pltpu.VMEM((_CH,), jnp.int32),
            pltpu.VMEM((_CH, _HID), jnp.float32),
            pltpu.VMEM((_CH, _HID), jnp.float32),
            pltpu.VMEM_SHARED((_NP, _HID), jnp.float32),
            pltpu.SemaphoreType.DMA,
        ],
    )
    return deg, scatter


def _dinv_of(cnt_blk):
    c = cnt_blk[0] + cnt_blk[1]
    return lax.rsqrt(c[:, 0:1] + 1.0)


def _l1_body(cnt_ref, x_ref, w1_ref, y_ref):
    dinv = _dinv_of(cnt_ref[...])
    xw = jnp.dot(x_ref[...], w1_ref[...], preferred_element_type=jnp.float32)
    y_ref[...] = xw * dinv


def _l2_body(cnt_ref, p_ref, y1_ref, w2_ref, b1_ref, y2_ref):
    dinv = _dinv_of(cnt_ref[...])
    h1 = jnp.maximum(dinv * (p_ref[0] + p_ref[1] + y1_ref[...]) + b1_ref[...], 0.0)
    y2_ref[...] = jnp.dot(h1, w2_ref[...], preferred_element_type=jnp.float32) * dinv


def _pool_body(cnt_ref, q_ref, y2_ref, batch_ref, b2_ref,
               wm1_ref, bm1_ref, wm2_ref, bm2_ref, out_ref, sums_ref, cg_ref):
    i = pl.program_id(0)
    dinv = _dinv_of(cnt_ref[...])
    h2 = jnp.maximum(dinv * (q_ref[0] + q_ref[1] + y2_ref[...]) + b2_ref[...], 0.0)

    b = batch_ref[0]
    gsel = jnp.broadcast_to(b, (_G, _BLK))
    grow = lax.broadcasted_iota(jnp.int32, (_G, _BLK), 0)
    node = i * _BLK + lax.broadcasted_iota(jnp.int32, (_G, _BLK), 1)
    mt = jnp.where((gsel == grow) & (node < _N), 1.0, 0.0)

    @pl.when(i == 0)
    def _():
        sums_ref[...] = jnp.zeros_like(sums_ref)
        cg_ref[...] = jnp.zeros_like(cg_ref)

    sums_ref[...] += jnp.dot(mt, h2, preferred_element_type=jnp.float32)
    cg_ref[...] += jnp.dot(mt, jnp.ones((_BLK, 1), jnp.float32),
                           preferred_element_type=jnp.float32)

    @pl.when(i == _NB - 1)
    def _():
        pooled = sums_ref[...] / jnp.maximum(cg_ref[...], 1.0)
        hid = jnp.maximum(
            jnp.dot(pooled, wm1_ref[...], preferred_element_type=jnp.float32)
            + bm1_ref[...], 0.0)
        out_ref[...] = (jnp.dot(hid, wm2_ref[...], preferred_element_type=jnp.float32)
                        + bm2_ref[...])


_cnt_spec = pl.BlockSpec((_NC, _BLK, 16), lambda i: (0, i, 0))
_row_spec = pl.BlockSpec((_BLK, _HID), lambda i: (i, 0))
_par_spec = pl.BlockSpec((_NC, _BLK, _HID), lambda i: (0, i, 0))


def _l1_call(cnt, xp, W1):
    return pl.pallas_call(
        _l1_body,
        grid=(_NB,),
        in_specs=[
            _cnt_spec,
            pl.BlockSpec((_BLK, _IN), lambda i: (i, 0)),
            pl.BlockSpec((_IN, _HID), lambda i: (0, 0)),
        ],
        out_specs=_row_spec,
        out_shape=jax.ShapeDtypeStruct((_NP, _HID), jnp.float32),
    )(cnt, xp, W1)


def _l2_call(cnt, p, y1, W2, b1):
    return pl.pallas_call(
        _l2_body,
        grid=(_NB,),
        in_specs=[
            _cnt_spec,
            _par_spec,
            _row_spec,
            pl.BlockSpec((_HID, _HID), lambda i: (0, 0)),
            pl.BlockSpec((1, _HID), lambda i: (0, 0)),
        ],
        out_specs=_row_spec,
        out_shape=jax.ShapeDtypeStruct((_NP, _HID), jnp.float32),
    )(cnt, p, y1, W2, b1)


def _pool_call(cnt, q, y2, batchp, b2, Wm1, bm1, Wm2, bm2):
    return pl.pallas_call(
        _pool_body,
        grid=(_NB,),
        in_specs=[
            _cnt_spec,
            _par_spec,
            _row_spec,
            pl.BlockSpec((1, 1, _BLK), lambda i: (i, 0, 0)),
            pl.BlockSpec((1, _HID), lambda i: (0, 0)),
            pl.BlockSpec((_HID, _HID), lambda i: (0, 0)),
            pl.BlockSpec((1, _HID), lambda i: (0, 0)),
            pl.BlockSpec((_HID, _OUTD), lambda i: (0, 0)),
            pl.BlockSpec((1, _OUTD), lambda i: (0, 0)),
        ],
        out_specs=pl.BlockSpec((_G, _OUTD), lambda i: (0, 0)),
        out_shape=jax.ShapeDtypeStruct((_G, _OUTD), jnp.float32),
        scratch_shapes=[
            pltpu.VMEM((_G, _HID), jnp.float32),
            pltpu.VMEM((_G, 1), jnp.float32),
        ],
    )(cnt, q, y2, batchp, b2, Wm1, bm1, Wm2, bm2)


def kernel(x, edge_index, batch, W1, b1, W2, b2, Wm1, bm1, Wm2, bm2):
    src = edge_index[0].astype(jnp.int32)
    dst = edge_index[1].astype(jnp.int32)
    srcp = jnp.concatenate([src, jnp.zeros((_EP - _E,), jnp.int32)])
    dstp = jnp.concatenate([dst, jnp.full((_EP - _E,), _DISCARD, jnp.int32)])
    xp = jnp.concatenate([x, jnp.zeros((_NP - _N, _IN), x.dtype)])
    batchp = jnp.concatenate(
        [batch.astype(jnp.int32), jnp.zeros((_NP - _N,), jnp.int32)]
    ).reshape(_NB, 1, _BLK)

    deg_kernel, scatter_kernel = _sc_kernels()
    cnt = deg_kernel(dstp)
    y1 = _l1_call(cnt, xp, W1)
    p = scatter_kernel(srcp, dstp, y1)
    y2 = _l2_call(cnt, p, y1, W2, b1.reshape(1, _HID))
    q = scatter_kernel(srcp, dstp, y2)
    return _pool_call(cnt, q, y2, batchp, b2.reshape(1, _HID),
                      Wm1, bm1.reshape(1, _HID), Wm2, bm2.reshape(1, _OUTD))
